# baseline (speedup 1.0000x reference)
import functools

import jax
import jax.numpy as jnp
from jax import lax
from jax.experimental import pallas as pl
from jax.experimental.pallas import tpu as pltpu
from jax.experimental.pallas import tpu_sc as plsc

B, H, W, C = 4, 112, 112, 192
OUT_H, OUT_W = H * 2, W * 2
TOTAL = B * OUT_H * OUT_W * C
N = B * H * W * C
CHUNK = 917_504
NCHUNK = TOTAL // CHUNK
NSUB = 16
PASSES = NCHUNK // 2
PER_TILE = N // NSUB
WIN = 6144
WROWS = WIN // 128
NWIN = PER_TILE // WIN
C16 = CHUNK // NSUB
RBLK = 128
CBUF = WIN + RBLK + 16


STG = 14336
NPIECE = C16 // STG


def _body(idx_hbm, upd_hbm, out_hbm, idxa, vala, idxb, valb, cidx, cval,
          zbuf, obuf, acc, sema, semb, semz):
    c = lax.axis_index("c")
    s = lax.axis_index("s")

    def zrow(i, carry):
        zbuf[pl.ds(i * 16, 16)] = jnp.zeros((16,), jnp.float32)
        return carry

    lax.fori_loop(0, STG // 16, zrow, 0, unroll=False)

    def _pass(p, carry):
            k = c * PASSES + p
            lo = k * CHUNK
            hi = lo + CHUNK

            for q in range(NPIECE):
                pltpu.async_copy(zbuf, acc.at[pl.ds(s * C16 + q * STG, STG)],
                                 semz)
            for q in range(NPIECE):
                pltpu.make_async_copy(
                    zbuf, acc.at[pl.ds(s * C16 + q * STG, STG)], semz).wait()
            plsc.subcore_barrier()

            def compact(iv, v, inr, fill):
                plsc.store_compressed(cidx.at[pl.ds(fill, 16)], iv - lo,
                                      mask=inr)
                plsc.store_compressed(cval.at[pl.ds(fill, 16)], v, mask=inr)
                pc = plsc.all_reduce_population_count(inr)
                return fill + pc[0]

            def drain(fill):
                fb = fill >> 7

                def fbody(fl):
                    pltpu.sync_copy(
                        cval.at[pl.ds(fl * RBLK, RBLK)],
                        acc.at[cidx.at[pl.ds(fl * RBLK, RBLK)]], add=True)
                    return fl + 1

                lax.while_loop(lambda fl: fl < fb, fbody, 0)
                for t in range(RBLK // 16):
                    src = fb * RBLK + t * 16
                    cidx[pl.ds(t * 16, 16)] = cidx[pl.ds(src, 16)]
                    cval[pl.ds(t * 16, 16)] = cval[pl.ds(src, 16)]
                return fill - fb * RBLK

            def start(w, ib, vb, sem):
                base = s * PER_TILE + w * WIN
                pltpu.async_copy(idx_hbm.at[pl.ds(base, WIN)], ib, sem)
                pltpu.async_copy(upd_hbm.at[pl.ds(base, WIN)], vb, sem)

            def wait(w, ib, vb, sem):
                base = s * PER_TILE + w * WIN
                pltpu.make_async_copy(
                    idx_hbm.at[pl.ds(base, WIN)], ib, sem).wait()
                pltpu.make_async_copy(
                    upd_hbm.at[pl.ds(base, WIN)], vb, sem).wait()

            def process(ib, vb, fill):
                def vrow(r, f2):
                    for cc in range(0, 128, 16):
                        off = r * 128 + cc
                        iv = ib[pl.ds(off, 16)]
                        v = vb[pl.ds(off, 16)]
                        inr = (iv >= lo) & (iv < hi)
                        f2 = compact(iv, v, inr, f2)
                    return f2

                fill = lax.fori_loop(0, WROWS, vrow, fill, unroll=False)
                return drain(fill)

            start(0, idxa, vala, sema)
            start(1, idxb, valb, semb)

            def dbl(t, fill):
                w = 2 * t
                wait(w, idxa, vala, sema)
                fill = process(idxa, vala, fill)

                @pl.when(w + 2 < NWIN)
                def _():
                    start(w + 2, idxa, vala, sema)

                wait(w + 1, idxb, valb, semb)
                fill = process(idxb, valb, fill)

                @pl.when(w + 3 < NWIN)
                def _():
                    start(w + 3, idxb, valb, semb)

                return fill

            fill = lax.fori_loop(0, NWIN // 2, dbl, 0, unroll=False)

            zi = jnp.zeros((16,), jnp.int32)
            zf = jnp.zeros((16,), jnp.float32)
            ones = jnp.ones((16,), jnp.bool_)
            for _ in range(RBLK // 16):
                fill = compact(lo + zi, zf, ones, fill)
            drain(fill)

            plsc.subcore_barrier()
            for q in range(NPIECE):
                off = s * C16 + q * STG
                pltpu.sync_copy(acc.at[pl.ds(off, STG)], obuf)
                pltpu.sync_copy(obuf, out_hbm.at[pl.ds(lo + off, STG)])
            return carry

    lax.fori_loop(0, PASSES, _pass, 0, unroll=False)


_scatter = functools.partial(
    pl.kernel,
    out_type=jax.ShapeDtypeStruct((TOTAL,), jnp.float32),
    mesh=plsc.VectorSubcoreMesh(core_axis_name="c", subcore_axis_name="s"),
    compiler_params=pltpu.CompilerParams(needs_layout_passes=False),
    scratch_types=[
        pltpu.VMEM((WIN,), jnp.int32),
        pltpu.VMEM((WIN,), jnp.float32),
        pltpu.VMEM((WIN,), jnp.int32),
        pltpu.VMEM((WIN,), jnp.float32),
        pltpu.VMEM((CBUF,), jnp.int32),
        pltpu.VMEM((CBUF,), jnp.float32),
        pltpu.VMEM((STG,), jnp.float32),
        pltpu.VMEM((STG,), jnp.float32),
        pltpu.VMEM_SHARED((CHUNK,), jnp.float32),
        pltpu.SemaphoreType.DMA,
        pltpu.SemaphoreType.DMA,
        pltpu.SemaphoreType.DMA,
    ],
)(_body)


@jax.jit
def kernel(updates, mask):
    idx = mask.astype(jnp.int32).reshape(N)
    upd = updates.reshape(N)
    out = _scatter(idx, upd)
    return out.reshape(-1, OUT_H, OUT_W, C)
